# baseline (speedup 1.0000x reference)
import functools

import jax
import jax.numpy as jnp
from jax import lax
from jax.experimental import pallas as pl
from jax.experimental.pallas import tpu as pltpu
from jax.experimental.pallas import tpu_sc as plsc

_N = 10000
_E = 160000
_B = 2
_D = 128
_NT = 16
_EPT = _E // _NT
_CH = 40
_NCHUNK = _EPT // _CH
_HD = _D // 2
_NPAD = 10240
_ROWS_PT = _NPAD // _NT
_WB = _CH
_TROWS = 7999


def _dot_t(x, w):
    return lax.dot_general(x, w, (((1,), (1,)), ((), ())),
                           preferred_element_type=jnp.float32)


def _node_tables_body(syn_ref, w1_ref, ai_ref, s_ref, aitab_ref):
    syn = syn_ref[...]
    s_ref[...] = _dot_t(syn, w1_ref[...])
    aitab_ref[...] = syn * ai_ref[...]


def _node_tables(syn, w1, ai_row):
    blk = 1000
    return pl.pallas_call(
        _node_tables_body,
        grid=(_N // blk,),
        in_specs=[
            pl.BlockSpec((blk, _D), lambda i: (i, 0)),
            pl.BlockSpec((_D, _D), lambda i: (0, 0)),
            pl.BlockSpec((1, _D), lambda i: (0, 0)),
        ],
        out_specs=[
            pl.BlockSpec((blk, _D), lambda i: (i, 0)),
            pl.BlockSpec((blk, _D), lambda i: (i, 0)),
        ],
        out_shape=[
            jax.ShapeDtypeStruct((_N, _D), jnp.float32),
            jax.ShapeDtypeStruct((_N, _D), jnp.float32),
        ],
    )(syn, w1, ai_row)


def _small_tables_body(tau_ref, edge_ref, pw_ref, pb_ref, zw_ref, zb_ref,
                       fw_ref, fb_ref, w2_ref, wnb_ref,
                       tp_ref, tz_ref, tf_ref, r_ref):
    tau = tau_ref[...]
    w2 = w2_ref[...]
    tp_ref[...] = _dot_t(_dot_t(tau, pw_ref[...]) + pb_ref[...], w2)
    tz_ref[...] = _dot_t(_dot_t(tau, zw_ref[...]) + zb_ref[...], w2)
    tf_ref[...] = _dot_t(_dot_t(tau, fw_ref[...]) + fb_ref[...], w2)
    r_ref[...] = _dot_t(edge_ref[...], w2) + wnb_ref[...]


def _small_tables(tau_p, edge_p, pw, pb, zw, zb, fw, fb, w2, wnb):
    tr = tau_p.shape[0]
    er = edge_p.shape[0]
    return pl.pallas_call(
        _small_tables_body,
        out_shape=[
            jax.ShapeDtypeStruct((tr, _D), jnp.float32),
            jax.ShapeDtypeStruct((tr, _D), jnp.float32),
            jax.ShapeDtypeStruct((tr, _D), jnp.float32),
            jax.ShapeDtypeStruct((er, _D), jnp.float32),
        ],
    )(tau_p, edge_p, pw, pb, zw, zb, fw, fb, w2, wnb)


def _finalize_body(num_ref, den_ref, out_ref):
    out_ref[...] = jax.nn.leaky_relu(num_ref[...] / (den_ref[...] + 1e-9))


def _finalize(num2d, den2d):
    blk = 1000
    w = num2d.shape[1]
    return pl.pallas_call(
        _finalize_body,
        grid=(_N // blk,),
        in_specs=[
            pl.BlockSpec((blk, w), lambda i: (i, 0)),
            pl.BlockSpec((blk, w), lambda i: (i, 0)),
        ],
        out_specs=pl.BlockSpec((blk, w), lambda i: (i, 0)),
        out_shape=jax.ShapeDtypeStruct((_N, w), jnp.float32),
    )(num2d, den2d)


def _edge_sc_kernel(s_tab, r_tab, t_tab, ai_tab, aj,
                    src, dst, rt, tidx_flat, zer):
    mesh = plsc.VectorSubcoreMesh(core_axis_name="c", subcore_axis_name="s")

    @functools.partial(
        pl.kernel,
        mesh=mesh,
        out_type=jax.ShapeDtypeStruct((4 * _NPAD, _D), jnp.float32),
        scratch_types=[
            pltpu.VMEM((_CH,), jnp.int32),
            pltpu.VMEM((_CH,), jnp.int32),
            pltpu.VMEM((_CH,), jnp.int32),
            pltpu.VMEM((_CH,), jnp.int32),
            pltpu.VMEM((_CH, _D), jnp.float32),
            pltpu.VMEM((_CH, _D), jnp.float32),
            pltpu.VMEM((_CH, _D), jnp.float32),
            pltpu.VMEM((_CH, _D), jnp.float32),
            pltpu.VMEM((_CH,), jnp.int32),
            pltpu.VMEM((_CH,), jnp.int32),
            pltpu.VMEM((_CH,), jnp.int32),
            pltpu.VMEM((_CH,), jnp.int32),
            pltpu.VMEM((_CH, _D), jnp.float32),
            pltpu.VMEM((_CH, _D), jnp.float32),
            pltpu.VMEM((_CH, _D), jnp.float32),
            pltpu.VMEM((_CH, _D), jnp.float32),
            pltpu.VMEM((_CH, _D), jnp.float32),
            pltpu.VMEM((_D,), jnp.float32),
            pltpu.VMEM_SHARED((_NPAD, _D), jnp.float32),
            pltpu.SemaphoreType.DMA,
            pltpu.SemaphoreType.DMA,
        ],
    )
    def k(sh, rh, th, aih, ajh,
          srch, dsth, rth, tixh, zerh, out,
          sv0, dv0, rv0, tv0, sr0, rr0, tr0, ar0,
          sv1, dv1, rv1, tv1, sr1, rr1, tr1, ar1,
          payload, aj_v, acc,
          g0, g1):
        cid = lax.axis_index("c")
        tid = lax.axis_index("s")

        pltpu.sync_copy(ajh, aj_v)

        bufs0 = (sv0, dv0, rv0, tv0, sr0, rr0, tr0, ar0, g0)
        bufs1 = (sv1, dv1, rv1, tv1, sr1, rr1, tr1, ar1, g1)

        def issue_chunk(ci, bufs):
            sv, dv, rv, tv, sr, rr, tr, ar, sem = bufs
            ebase = pl.multiple_of(tid * _EPT + ci * _CH, _CH)
            tbase = pl.multiple_of(cid * _E + ebase, _CH)
            pltpu.sync_copy(srch.at[pl.ds(ebase, _CH)], sv)
            pltpu.sync_copy(dsth.at[pl.ds(ebase, _CH)], dv)
            pltpu.sync_copy(rth.at[pl.ds(ebase, _CH)], rv)
            pltpu.sync_copy(tixh.at[pl.ds(tbase, _CH)], tv)
            pltpu.async_copy(sh.at[sv], sr, sem)
            pltpu.async_copy(rh.at[rv], rr, sem)
            pltpu.async_copy(th.at[tv], tr, sem)
            pltpu.async_copy(aih.at[dv], ar, sem)

        def wait_chunk(bufs):
            sv, dv, rv, tv, sr, rr, tr, ar, sem = bufs
            pltpu.make_async_copy(sh.at[sv], sr, sem).wait()
            pltpu.make_async_copy(rh.at[rv], rr, sem).wait()
            pltpu.make_async_copy(th.at[tv], tr, sem).wait()
            pltpu.make_async_copy(aih.at[dv], ar, sem).wait()

        def run_pass(h):
            pltpu.sync_copy(zerh, acc.at[pl.ds(tid * _ROWS_PT, _ROWS_PT)])
            plsc.subcore_barrier()

            aj_regs = [aj_v[pl.ds(h * _HD + u * 16, 16)] for u in range(4)]

            def compute_scatter(bufs):
                sv, dv, rv, tv, sr, rr, tr, ar, sem = bufs

                @plsc.parallel_loop(0, _CH, unroll=2)
                def edge_body(j):
                    for u in range(4):
                        sl = pl.ds(h * _HD + u * 16, 16)
                        g = sr[j, sl] + rr[j, sl] + tr[j, sl]
                        x = ar[j, sl] * g * aj_regs[u]
                        sc = jnp.where(x > 0, x, 0.01 * x)
                        ex = jnp.exp(sc)
                        payload[j, pl.ds(u * 16, 16)] = ex * g
                        payload[j, pl.ds(_HD + u * 16, 16)] = ex
                pltpu.sync_copy(payload, acc.at[dv], add=True)

            issue_chunk(0, bufs0)

            def j2_body(j2, carry):
                issue_chunk(2 * j2 + 1, bufs1)
                wait_chunk(bufs0)
                compute_scatter(bufs0)

                @pl.when(j2 < _NCHUNK // 2 - 1)
                def _():
                    issue_chunk(2 * j2 + 2, bufs0)

                wait_chunk(bufs1)
                compute_scatter(bufs1)
                return carry

            lax.fori_loop(0, _NCHUNK // 2, j2_body, 0)
            plsc.subcore_barrier()

            obase = (cid * 2 + h) * _NPAD + tid * _ROWS_PT

            def wb_body(kk, carry):
                rr = tid * _ROWS_PT + kk * _WB
                pltpu.sync_copy(acc.at[pl.ds(rr, _WB)], payload)
                pltpu.sync_copy(payload, out.at[pl.ds(obase + kk * _WB, _WB)])
                return carry

            lax.fori_loop(0, _ROWS_PT // _WB, wb_body, 0)
            plsc.subcore_barrier()

        run_pass(0)
        run_pass(1)

    return k(s_tab, r_tab, t_tab, ai_tab, aj,
             src, dst, rt, tidx_flat, zer)


def kernel(node_idx, edge_index, relation_type, edge_time, batch_time,
           syn_table, edge_table, tau_table, W_n_w, W_n_b,
           past_w, past_b, pres_w, pres_b, fut_w, fut_b,
           attn_i, attn_j):
    src = edge_index[0]
    dst = edge_index[1]
    w1 = W_n_w[:, :_D]
    w2 = W_n_w[:, _D:]

    syn = jnp.take(syn_table, node_idx, axis=0)
    ai_row = attn_i.reshape(1, _D)
    aj_row = attn_j.reshape(_D)

    s_tab, ai_tab = _node_tables(syn, w1, ai_row)

    tau_p = tau_table[:4001]
    tp, tz, tf, r_tab = _small_tables(
        tau_p, edge_table,
        past_w, past_b.reshape(1, _D),
        pres_w, pres_b.reshape(1, _D),
        fut_w, fut_b.reshape(1, _D),
        w2, W_n_b.reshape(1, _D))
    t_tab = jnp.concatenate([tp[2:4001][::-1], tz[1:2], tf[2:4001]], axis=0)

    tidx = (edge_time[None, :] - batch_time[:, None] + 3999).astype(jnp.int32)
    tidx_flat = tidx.reshape(_B * _E)

    zer = jnp.zeros((_ROWS_PT, _D), jnp.float32)

    acc = _edge_sc_kernel(
        s_tab, r_tab, t_tab, ai_tab,
        aj_row, src, dst, relation_type, tidx_flat, zer)

    acc = acc.reshape(_B, 2, _NPAD, _D)[:, :, :_N, :]
    num = jnp.concatenate([acc[:, 0, :, :_HD], acc[:, 1, :, :_HD]], axis=-1)
    den = jnp.concatenate([acc[:, 0, :, _HD:], acc[:, 1, :, _HD:]], axis=-1)
    num2d = num.transpose(1, 0, 2).reshape(_N, _B * _D)
    den2d = den.transpose(1, 0, 2).reshape(_N, _B * _D)
    out = _finalize(num2d, den2d)
    return out.reshape(_N, _B, _D)


# speedup vs baseline: 1.5055x; 1.5055x over previous
import functools

import jax
import jax.numpy as jnp
from jax import lax
from jax.experimental import pallas as pl
from jax.experimental.pallas import tpu as pltpu
from jax.experimental.pallas import tpu_sc as plsc

_N = 10000
_E = 160000
_B = 2
_D = 128
_NT = 16
_EPT = _E // _NT
_CH = 40
_NCHUNK = _EPT // _CH
_HD = _D // 2
_NPAD = 10240
_ROWS_PT = _NPAD // _NT
_WB = _CH
_TROWS = 7999


def _dot_t(x, w):
    return lax.dot_general(x, w, (((1,), (1,)), ((), ())),
                           preferred_element_type=jnp.float32)


def _node_tables_body(syn_ref, w1_ref, ai_ref, s_ref, aitab_ref):
    syn = syn_ref[...]
    s_ref[...] = _dot_t(syn, w1_ref[...])
    aitab_ref[...] = syn * ai_ref[...]


def _node_tables(syn, w1, ai_row):
    blk = 1000
    return pl.pallas_call(
        _node_tables_body,
        grid=(_N // blk,),
        in_specs=[
            pl.BlockSpec((blk, _D), lambda i: (i, 0)),
            pl.BlockSpec((_D, _D), lambda i: (0, 0)),
            pl.BlockSpec((1, _D), lambda i: (0, 0)),
        ],
        out_specs=[
            pl.BlockSpec((blk, _D), lambda i: (i, 0)),
            pl.BlockSpec((blk, _D), lambda i: (i, 0)),
        ],
        out_shape=[
            jax.ShapeDtypeStruct((_N, _D), jnp.float32),
            jax.ShapeDtypeStruct((_N, _D), jnp.float32),
        ],
    )(syn, w1, ai_row)


def _small_tables_body(tau_ref, edge_ref, pw_ref, pb_ref, zw_ref, zb_ref,
                       fw_ref, fb_ref, w2_ref, wnb_ref,
                       tp_ref, tz_ref, tf_ref, r_ref):
    tau = tau_ref[...]
    w2 = w2_ref[...]
    tp_ref[...] = _dot_t(_dot_t(tau, pw_ref[...]) + pb_ref[...], w2)
    tz_ref[...] = _dot_t(_dot_t(tau, zw_ref[...]) + zb_ref[...], w2)
    tf_ref[...] = _dot_t(_dot_t(tau, fw_ref[...]) + fb_ref[...], w2)
    r_ref[...] = _dot_t(edge_ref[...], w2) + wnb_ref[...]


def _small_tables(tau_p, edge_p, pw, pb, zw, zb, fw, fb, w2, wnb):
    tr = tau_p.shape[0]
    er = edge_p.shape[0]
    return pl.pallas_call(
        _small_tables_body,
        out_shape=[
            jax.ShapeDtypeStruct((tr, _D), jnp.float32),
            jax.ShapeDtypeStruct((tr, _D), jnp.float32),
            jax.ShapeDtypeStruct((tr, _D), jnp.float32),
            jax.ShapeDtypeStruct((er, _D), jnp.float32),
        ],
    )(tau_p, edge_p, pw, pb, zw, zb, fw, fb, w2, wnb)


def _finalize_body(num_ref, den_ref, out_ref):
    out_ref[...] = jax.nn.leaky_relu(num_ref[...] / (den_ref[...] + 1e-9))


def _finalize(num2d, den2d):
    blk = 1000
    w = num2d.shape[1]
    return pl.pallas_call(
        _finalize_body,
        grid=(_N // blk,),
        in_specs=[
            pl.BlockSpec((blk, w), lambda i: (i, 0)),
            pl.BlockSpec((blk, w), lambda i: (i, 0)),
        ],
        out_specs=pl.BlockSpec((blk, w), lambda i: (i, 0)),
        out_shape=jax.ShapeDtypeStruct((_N, w), jnp.float32),
    )(num2d, den2d)


def _edge_sc_kernel(s_tab, r_tab, t_tab, ai_tab, aj,
                    src, dst, rt, tidx_flat, zer):
    mesh = plsc.VectorSubcoreMesh(core_axis_name="c", subcore_axis_name="s")

    @functools.partial(
        pl.kernel,
        mesh=mesh,
        out_type=jax.ShapeDtypeStruct((4 * _NPAD, _D), jnp.float32),
        scratch_types=[
            pltpu.VMEM((_CH,), jnp.int32),
            pltpu.VMEM((_CH,), jnp.int32),
            pltpu.VMEM((_CH,), jnp.int32),
            pltpu.VMEM((_CH,), jnp.int32),
            pltpu.VMEM((_CH, _D), jnp.float32),
            pltpu.VMEM((_CH, _D), jnp.float32),
            pltpu.VMEM((_CH, _D), jnp.float32),
            pltpu.VMEM((_CH, _D), jnp.float32),
            pltpu.VMEM((_CH,), jnp.int32),
            pltpu.VMEM((_CH,), jnp.int32),
            pltpu.VMEM((_CH,), jnp.int32),
            pltpu.VMEM((_CH,), jnp.int32),
            pltpu.VMEM((_CH, _D), jnp.float32),
            pltpu.VMEM((_CH, _D), jnp.float32),
            pltpu.VMEM((_CH, _D), jnp.float32),
            pltpu.VMEM((_CH, _D), jnp.float32),
            pltpu.VMEM((_CH, _D), jnp.float32),
            pltpu.VMEM((_D,), jnp.float32),
            pltpu.VMEM_SHARED((_NPAD, _D), jnp.float32),
            pltpu.SemaphoreType.DMA,
            pltpu.SemaphoreType.DMA,
        ],
    )
    def k(sh, rh, th, aih, ajh,
          srch, dsth, rth, tixh, zerh, out,
          sv0, dv0, rv0, tv0, sr0, rr0, tr0, ar0,
          sv1, dv1, rv1, tv1, sr1, rr1, tr1, ar1,
          payload, aj_v, acc,
          g0, g1):
        cid = lax.axis_index("c")
        tid = lax.axis_index("s")

        pltpu.sync_copy(ajh, aj_v)

        bufs0 = (sv0, dv0, rv0, tv0, sr0, rr0, tr0, ar0, g0)
        bufs1 = (sv1, dv1, rv1, tv1, sr1, rr1, tr1, ar1, g1)

        def issue_chunk(ci, bufs):
            sv, dv, rv, tv, sr, rr, tr, ar, sem = bufs
            ebase = pl.multiple_of(tid * _EPT + ci * _CH, _CH)
            tbase = pl.multiple_of(cid * _E + ebase, _CH)
            pltpu.async_copy(srch.at[pl.ds(ebase, _CH)], sv, sem)
            pltpu.async_copy(dsth.at[pl.ds(ebase, _CH)], dv, sem)
            pltpu.async_copy(rth.at[pl.ds(ebase, _CH)], rv, sem)
            pltpu.async_copy(tixh.at[pl.ds(tbase, _CH)], tv, sem)
            pltpu.make_async_copy(srch.at[pl.ds(ebase, _CH)], sv, sem).wait()
            pltpu.make_async_copy(dsth.at[pl.ds(ebase, _CH)], dv, sem).wait()
            pltpu.make_async_copy(rth.at[pl.ds(ebase, _CH)], rv, sem).wait()
            pltpu.make_async_copy(tixh.at[pl.ds(tbase, _CH)], tv, sem).wait()
            pltpu.async_copy(sh.at[sv], sr, sem)
            pltpu.async_copy(rh.at[rv], rr, sem)
            pltpu.async_copy(th.at[tv], tr, sem)
            pltpu.async_copy(aih.at[dv], ar, sem)

        def wait_chunk(bufs):
            sv, dv, rv, tv, sr, rr, tr, ar, sem = bufs
            pltpu.make_async_copy(sh.at[sv], sr, sem).wait()
            pltpu.make_async_copy(rh.at[rv], rr, sem).wait()
            pltpu.make_async_copy(th.at[tv], tr, sem).wait()
            pltpu.make_async_copy(aih.at[dv], ar, sem).wait()

        def run_pass(h):
            pltpu.sync_copy(zerh, acc.at[pl.ds(tid * _ROWS_PT, _ROWS_PT)])
            plsc.subcore_barrier()

            aj_regs = [aj_v[pl.ds(h * _HD + u * 16, 16)] for u in range(4)]

            def compute_scatter(bufs):
                sv, dv, rv, tv, sr, rr, tr, ar, sem = bufs

                def edge_body(j, carry2):
                    for u in range(4):
                        sl = pl.ds(h * _HD + u * 16, 16)
                        g = sr[j, sl] + rr[j, sl] + tr[j, sl]
                        x = ar[j, sl] * g * aj_regs[u]
                        sc = jnp.where(x > 0, x, 0.01 * x)
                        ex = jnp.exp(sc)
                        payload[j, pl.ds(u * 16, 16)] = ex * g
                        payload[j, pl.ds(_HD + u * 16, 16)] = ex
                    return carry2

                lax.fori_loop(0, _CH, edge_body, 0)
                pltpu.sync_copy(payload, acc.at[dv], add=True)

            issue_chunk(0, bufs0)

            def j2_body(j2, carry):
                issue_chunk(2 * j2 + 1, bufs1)
                wait_chunk(bufs0)
                compute_scatter(bufs0)

                @pl.when(j2 < _NCHUNK // 2 - 1)
                def _():
                    issue_chunk(2 * j2 + 2, bufs0)

                wait_chunk(bufs1)
                compute_scatter(bufs1)
                return carry

            lax.fori_loop(0, _NCHUNK // 2, j2_body, 0)
            plsc.subcore_barrier()

            obase = (cid * 2 + h) * _NPAD + tid * _ROWS_PT

            def wb_body(kk, carry):
                rr = tid * _ROWS_PT + kk * _WB
                pltpu.sync_copy(acc.at[pl.ds(rr, _WB)], payload)
                pltpu.sync_copy(payload, out.at[pl.ds(obase + kk * _WB, _WB)])
                return carry

            lax.fori_loop(0, _ROWS_PT // _WB, wb_body, 0)
            plsc.subcore_barrier()

        run_pass(0)
        run_pass(1)

    return k(s_tab, r_tab, t_tab, ai_tab, aj,
             src, dst, rt, tidx_flat, zer)


def kernel(node_idx, edge_index, relation_type, edge_time, batch_time,
           syn_table, edge_table, tau_table, W_n_w, W_n_b,
           past_w, past_b, pres_w, pres_b, fut_w, fut_b,
           attn_i, attn_j):
    src = edge_index[0]
    dst = edge_index[1]
    w1 = W_n_w[:, :_D]
    w2 = W_n_w[:, _D:]

    syn = jnp.take(syn_table, node_idx, axis=0)
    ai_row = attn_i.reshape(1, _D)
    aj_row = attn_j.reshape(_D)

    s_tab, ai_tab = _node_tables(syn, w1, ai_row)

    tau_p = tau_table[:4001]
    tp, tz, tf, r_tab = _small_tables(
        tau_p, edge_table,
        past_w, past_b.reshape(1, _D),
        pres_w, pres_b.reshape(1, _D),
        fut_w, fut_b.reshape(1, _D),
        w2, W_n_b.reshape(1, _D))
    t_tab = jnp.concatenate([tp[2:4001][::-1], tz[1:2], tf[2:4001]], axis=0)

    tidx = (edge_time[None, :] - batch_time[:, None] + 3999).astype(jnp.int32)
    tidx_flat = tidx.reshape(_B * _E)

    zer = jnp.zeros((_ROWS_PT, _D), jnp.float32)

    acc = _edge_sc_kernel(
        s_tab, r_tab, t_tab, ai_tab,
        aj_row, src, dst, relation_type, tidx_flat, zer)

    acc = acc.reshape(_B, 2, _NPAD, _D)[:, :, :_N, :]
    num = jnp.concatenate([acc[:, 0, :, :_HD], acc[:, 1, :, :_HD]], axis=-1)
    den = jnp.concatenate([acc[:, 0, :, _HD:], acc[:, 1, :, _HD:]], axis=-1)
    num2d = num.transpose(1, 0, 2).reshape(_N, _B * _D)
    den2d = den.transpose(1, 0, 2).reshape(_N, _B * _D)
    out = _finalize(num2d, den2d)
    return out.reshape(_N, _B, _D)
